# baseline (speedup 1.0000x reference)
import functools

import jax
import jax.numpy as jnp
from jax import lax
from jax.experimental import pallas as pl
from jax.experimental.pallas import tpu as pltpu
from jax.experimental.pallas import tpu_sc as plsc

EMB = 64
SCALE = 8.0
VOCAB = 1000000

NC = 2
NS = 16
NW = NC * NS
B = 4096 * 200
PER_W = B // NW
CH = 512
NCHUNK = PER_W // CH
G = 128
NG = CH // G

AW = 2048
JT, JL = 25, 8
TI, IL = 32, 128
TK, KL = 8, 8


def _tab_fmt_body(x_ref, o_ref):
    x = x_ref[...]
    y = (x.T * SCALE).reshape(AW // 2, 2, EMB)
    o_ref[:, 0:EMB] = y[:, 0, :]
    o_ref[:, EMB:128] = y[:, 1, :]


_tab_fmt = pl.pallas_call(
    _tab_fmt_body,
    grid=((VOCAB + AW - 1) // AW,),
    in_specs=[pl.BlockSpec((EMB, AW), lambda i: (0, i))],
    out_specs=pl.BlockSpec((AW // 2, 128), lambda i: (i, 0)),
    out_shape=jax.ShapeDtypeStruct((VOCAB // 2, 128), jnp.float32),
)


def _emb_body(tok_hbm, tab_hbm, out_hbm,
              idx0, idx1, rows0, rows1, gsem0, gsem1, wsem0, wsem1):
    w = lax.axis_index("s") * NC + lax.axis_index("c")
    base_w = w * PER_W
    idx = (idx0, idx1)
    rows = (rows0, rows1)
    gsem = (gsem0, gsem1)
    wsem = (wsem0, wsem1)

    def fire_gather(ci, b):
        base = base_w + ci * CH
        pltpu.sync_copy(tok_hbm.at[pl.ds(base, CH)], idx[b])
        for j in range(NG):
            pltpu.async_copy(
                tab_hbm.at[idx[b].at[pl.ds(j * G, G)]],
                rows[b].at[pl.ds(j * G, G), :],
                gsem[b],
            )

    def drain_gather(b):
        pltpu.make_async_copy(tab_hbm.at[pl.ds(0, CH)], rows[b], gsem[b]).wait()

    def drain_write(b, ci):
        pltpu.make_async_copy(
            rows[b], out_hbm.at[pl.ds(base_w + ci * CH, CH)], wsem[b]).wait()

    fire_gather(0, 0)

    def outer(co, carry):
        for b in range(2):
            ci = co * 2 + b

            @pl.when(ci >= 1)
            def _():
                drain_write(1 - b, ci - 1)

            @pl.when(ci + 1 < NCHUNK)
            def _():
                fire_gather(ci + 1, 1 - b)

            drain_gather(b)
            pltpu.async_copy(
                rows[b], out_hbm.at[pl.ds(base_w + ci * CH, CH)], wsem[b])
        return carry

    lax.fori_loop(0, NCHUNK // 2, outer, 0)
    drain_write(1, NCHUNK - 1)


@functools.partial(
    pl.kernel,
    mesh=plsc.VectorSubcoreMesh(core_axis_name="c", subcore_axis_name="s"),
    out_type=jax.ShapeDtypeStruct((B, EMB), jnp.float32),
    scratch_types=[
        pltpu.VMEM((CH,), jnp.int32),
        pltpu.VMEM((CH,), jnp.int32),
        pltpu.VMEM((CH, EMB), jnp.float32),
        pltpu.VMEM((CH, EMB), jnp.float32),
        pltpu.SemaphoreType.DMA,
        pltpu.SemaphoreType.DMA,
        pltpu.SemaphoreType.DMA,
        pltpu.SemaphoreType.DMA,
    ],
    compiler_params=pltpu.CompilerParams(use_tc_tiling_on_sc=False),
)
def _emb_kernel(tok_hbm, tab_hbm, out_hbm, *scratch):
    _emb_body(tok_hbm, tab_hbm, out_hbm, *scratch)


def _out_fmt_body(x_ref, o_ref):
    x = x_ref[...]
    o_ref[...] = jnp.transpose(x, (1, 2, 0)).reshape(JL, TK, 1, KL, IL)


_out_fmt = pl.pallas_call(
    _out_fmt_body,
    grid=(JT, TI),
    in_specs=[pl.BlockSpec((IL, JL, EMB), lambda jt, t: (t, jt, 0))],
    out_specs=pl.BlockSpec(
        (JL, TK, 1, KL, IL), lambda jt, t: (jt, 0, t, 0, 0)),
    out_shape=jax.ShapeDtypeStruct((JT * JL, TK, TI, KL, IL), jnp.float32),
)


def kernel(tokens, table):
    tok_flat = tokens.astype(jnp.int32).reshape(-1)
    tab_lin = _tab_fmt(table.T).reshape(VOCAB, EMB)
    rows = _emb_kernel(tok_flat, tab_lin)
    out5 = _out_fmt(rows.reshape(4096, 200, EMB))
    return out5.transpose(2, 4, 0, 1, 3).reshape(4096, 200, EMB)


# speedup vs baseline: 2.2464x; 2.2464x over previous
import functools

import jax
import jax.numpy as jnp
from jax import lax
from jax.experimental import pallas as pl
from jax.experimental.pallas import tpu as pltpu
from jax.experimental.pallas import tpu_sc as plsc

EMB = 64
SCALE = 8.0

B = 4096 * 200
NC = 2
NS = 16
NW = NC * NS
PER_W = B // NW
C = 512
NCHUNK = PER_W // C
G = 128
NG = C // G


def _emb_body(tok_hbm, tab_hbm, out_hbm,
              idx0, idx1, rows0, rows1, gsem0, gsem1, wsem0, wsem1):
    wid = lax.axis_index("s") * NC + lax.axis_index("c")
    base_w = wid * PER_W
    idx = (idx0, idx1)
    rows = (rows0, rows1)
    gsem = (gsem0, gsem1)
    wsem = (wsem0, wsem1)

    def fire_gather(ci, b):
        base = base_w + ci * C
        pltpu.sync_copy(tok_hbm.at[pl.ds(base, C)], idx[b])
        for j in range(NG):
            pltpu.async_copy(
                tab_hbm.at[idx[b].at[pl.ds(j * G, G)]],
                rows[b].at[pl.ds(j * G, G), :],
                gsem[b],
            )

    def drain_gather(b):
        pltpu.make_async_copy(tab_hbm.at[pl.ds(0, C)], rows[b], gsem[b]).wait()

    def drain_write(b, ci):
        pltpu.make_async_copy(
            rows[b], out_hbm.at[pl.ds(base_w + ci * C, C)], wsem[b]).wait()

    def scale(b):
        rb = rows[b]

        @plsc.parallel_loop(0, C, unroll=4)
        def _(r):
            for c4 in range(EMB // 16):
                sl = pl.ds(c4 * 16, 16)
                rb[r, sl] = rb[r, sl] * SCALE

    fire_gather(0, 0)

    def outer(co, carry):
        for b in range(2):
            ci = co * 2 + b

            @pl.when(ci >= 1)
            def _():
                drain_write(1 - b, ci - 1)

            @pl.when(ci + 1 < NCHUNK)
            def _():
                fire_gather(ci + 1, 1 - b)

            drain_gather(b)
            scale(b)
            pltpu.async_copy(
                rows[b], out_hbm.at[pl.ds(base_w + ci * C, C)], wsem[b])
        return carry

    lax.fori_loop(0, NCHUNK // 2, outer, 0)
    drain_write(1, NCHUNK - 1)


@functools.partial(
    pl.kernel,
    mesh=plsc.VectorSubcoreMesh(core_axis_name="c", subcore_axis_name="s"),
    out_type=jax.ShapeDtypeStruct((B, EMB), jnp.float32),
    scratch_types=[
        pltpu.VMEM((C,), jnp.int32),
        pltpu.VMEM((C,), jnp.int32),
        pltpu.VMEM((C, EMB), jnp.float32),
        pltpu.VMEM((C, EMB), jnp.float32),
        pltpu.SemaphoreType.DMA,
        pltpu.SemaphoreType.DMA,
        pltpu.SemaphoreType.DMA,
        pltpu.SemaphoreType.DMA,
    ],
    compiler_params=pltpu.CompilerParams(use_tc_tiling_on_sc=False),
)
def _emb_kernel(tok_hbm, tab_hbm, out_hbm,
                idx0, idx1, rows0, rows1, gsem0, gsem1, wsem0, wsem1):
    _emb_body(tok_hbm, tab_hbm, out_hbm,
              idx0, idx1, rows0, rows1, gsem0, gsem1, wsem0, wsem1)


def kernel(tokens, table):
    tok_flat = tokens.reshape(-1).astype(jnp.int32)
    out = _emb_kernel(tok_flat, table)
    return out.reshape(tokens.shape + (EMB,))
